# baseline (speedup 1.0000x reference)
import jax
import jax.numpy as jnp
from jax.experimental import pallas as pl
from jax.experimental.pallas import tpu as pltpu

_R = 14336


def _body(nc_ref, t_ref, x_ref, out_ref, nf_ref):
    t = t_ref[...]
    c0 = nc_ref[0, 0]
    c1 = nc_ref[1, 0]
    c2 = nc_ref[2, 0]
    c3 = nc_ref[3, 0]
    f = jnp.where(t == 0, c0, jnp.where(t == 1, c1, jnp.where(t == 2, c2, c3)))
    nf_ref[...] = f
    f_col = f.reshape(_R, 1)
    out_ref[...] = x_ref[...] * f_col


def kernel(node_features, atom_types, norm_const):
    n, d = node_features.shape
    g = -(-n // _R)
    t32 = atom_types.astype(jnp.int32)
    out_features, nf = pl.pallas_call(
        _body,
        grid=(g,),
        in_specs=[
            pl.BlockSpec(memory_space=pltpu.SMEM),
            pl.BlockSpec((_R,), lambda i: (i,)),
            pl.BlockSpec((_R, d), lambda i: (i, 0)),
        ],
        out_specs=[
            pl.BlockSpec((_R, d), lambda i: (i, 0)),
            pl.BlockSpec((_R,), lambda i: (i,)),
        ],
        out_shape=[
            jax.ShapeDtypeStruct((n, d), jnp.float32),
            jax.ShapeDtypeStruct((n,), jnp.float32),
        ],
        compiler_params=pltpu.CompilerParams(
            dimension_semantics=("arbitrary",),
        ),
    )(norm_const, t32, node_features)
    return out_features, nf.reshape(n, 1)


# speedup vs baseline: 1.0070x; 1.0070x over previous
import jax
import jax.numpy as jnp
from jax.experimental import pallas as pl
from jax.experimental.pallas import tpu as pltpu

_R = 13312


def _body(nc_ref, t_ref, x_ref, out_ref, nf_ref):
    t = t_ref[...]
    c0 = nc_ref[0, 0]
    c1 = nc_ref[1, 0]
    c2 = nc_ref[2, 0]
    c3 = nc_ref[3, 0]
    f = jnp.where(t == 0, c0, jnp.where(t == 1, c1, jnp.where(t == 2, c2, c3)))
    nf_ref[...] = f
    f_col = f.reshape(_R, 1)
    out_ref[...] = x_ref[...] * f_col


def kernel(node_features, atom_types, norm_const):
    n, d = node_features.shape
    g = -(-n // _R)
    t32 = atom_types.astype(jnp.int32)
    out_features, nf = pl.pallas_call(
        _body,
        grid=(g,),
        in_specs=[
            pl.BlockSpec(memory_space=pltpu.SMEM),
            pl.BlockSpec((_R,), lambda i: (i,)),
            pl.BlockSpec((_R, d), lambda i: (i, 0)),
        ],
        out_specs=[
            pl.BlockSpec((_R, d), lambda i: (i, 0)),
            pl.BlockSpec((_R,), lambda i: (i,)),
        ],
        out_shape=[
            jax.ShapeDtypeStruct((n, d), jnp.float32),
            jax.ShapeDtypeStruct((n,), jnp.float32),
        ],
        compiler_params=pltpu.CompilerParams(
            dimension_semantics=("arbitrary",),
        ),
    )(norm_const, t32, node_features)
    return out_features, nf.reshape(n, 1)
